# baseline (speedup 1.0000x reference)
import functools

import jax
import jax.numpy as jnp
from jax import lax
from jax.experimental import pallas as pl
from jax.experimental.pallas import tpu as pltpu
from jax.experimental.pallas import tpu_sc as plsc

M_ = 100
MP = 128
K2_ = 9
KDIM_ = 256
VDIM_ = 3
TW = 384
H_ = 64
W_ = 64
HW_ = H_ * W_
PAD = 72
KP_ROWS = HW_ + 2 * PAD
TROWS = HW_ + 8
RBLK = 512
NC = 2
NS = 16
NW = NC * NS


def _select_body(k_ref, v_ref, mk_ref, nbr_ref, tbl_ref, kp):
    b = pl.program_id(0)
    tbl_ref[0:HW_, 0:KDIM_] = k_ref[0]
    tbl_ref[0:HW_, KDIM_:TW] = jnp.zeros((HW_, TW - KDIM_),
                                         jnp.float32)
    tbl_ref[0:HW_, KDIM_:KDIM_ + VDIM_] = v_ref[0]
    tbl_ref[HW_:, :] = jnp.zeros((TROWS - HW_, TW), jnp.float32)

    kp[0:PAD, :] = jnp.zeros((PAD, KDIM_), jnp.float32)
    kp[PAD + HW_:, :] = jnp.zeros((PAD, KDIM_), jnp.float32)
    kp[PAD:PAD + HW_, :] = k_ref[0]

    xcol = jax.lax.broadcasted_iota(jnp.int32, (HW_, 1), 0) % W_

    acc = jnp.zeros((HW_, MP), jnp.float32)
    for t in range(K2_):
        dy, dx = t // 3 - 1, t % 3 - 1
        start = PAD + W_ * dy + dx
        sh = kp[start:start + HW_, :]
        if dx == -1:
            sh = sh * (xcol >= 1).astype(jnp.float32)
        elif dx == 1:
            sh = sh * (xcol <= W_ - 2).astype(jnp.float32)
        w = mk_ref[0, :, t, :]
        acc = acc + jax.lax.dot_general(
            sh, w, (((1,), (1,)), ((), ())),
            preferred_element_type=jnp.float32)

    mcol = jax.lax.broadcasted_iota(jnp.int32, (HW_, MP), 1)
    lm = jnp.where(mcol < M_, acc, -1e30)
    lmax = jnp.max(lm, axis=1, keepdims=True)
    denom = jnp.sum(jnp.exp(lm - lmax), axis=1, keepdims=True)
    val = 1.0 / denom
    valT = jnp.transpose(val)

    cnt_blocks = []
    for i in range(HW_ // RBLK):
        vi = val[i * RBLK:(i + 1) * RBLK, :]
        pio = (jax.lax.broadcasted_iota(jnp.int32, (RBLK, 1), 0)
               + i * RBLK)
        qio = jax.lax.broadcasted_iota(jnp.int32, (RBLK, HW_), 1)
        less = valT < vi
        tie = (valT == vi) & (qio < pio)
        cnt_blocks.append(jnp.sum(jnp.where(less | tie, 1.0, 0.0),
                                  axis=1, keepdims=True))
    rank_col = jnp.concatenate(cnt_blocks, axis=0)

    r_io = jax.lax.broadcasted_iota(jnp.int32, (HW_, MP), 1).astype(
        jnp.float32)
    q_io = jax.lax.broadcasted_iota(jnp.int32, (HW_, MP), 0).astype(
        jnp.float32)
    hit = rank_col == r_io
    srcT = jnp.sum(jnp.where(hit, q_io, 0.0), axis=0, keepdims=True)
    src = srcT.astype(jnp.int32)
    xs = src % W_

    for t in range(K2_):
        dy, dx = t // 3 - 1, t % 3 - 1
        nbr = src + (W_ * dy + dx) + b * TROWS
        ok_y = jnp.logical_and(src + W_ * dy >= 0,
                               src + W_ * dy < HW_)
        if dx == -1:
            valid = jnp.logical_and(ok_y, xs >= 1)
        elif dx == 1:
            valid = jnp.logical_and(ok_y, xs <= W_ - 2)
        else:
            valid = ok_y
        nbr = jnp.where(valid, nbr, b * TROWS + HW_)
        nbr_ref[0, pl.ds(t, 1), :] = nbr


def _sc_gather_body(ch, nbr_hbm, tbl_hbm, out_hbm, idx_v, rows, sem):
    wid = lax.axis_index("s") * NC + lax.axis_index("c")
    base = wid * ch
    pltpu.sync_copy(nbr_hbm.at[pl.ds(base, ch)], idx_v)
    pltpu.async_copy(tbl_hbm.at[idx_v], rows, sem).wait()
    pltpu.sync_copy(rows, out_hbm.at[pl.ds(base, ch)])


def kernel(k, v, rkn_score, m_k, m_v, m_u):
    del rkn_score, m_v, m_u
    B = k.shape[0]
    mk_pad = jnp.pad(m_k, ((0, 0), (0, MP - M_), (0, 0), (0, 0)))

    nbr, tbl = pl.pallas_call(
        _select_body,
        grid=(B,),
        in_specs=[
            pl.BlockSpec((1, HW_, KDIM_), lambda b: (b, 0, 0)),
            pl.BlockSpec((1, HW_, VDIM_), lambda b: (b, 0, 0)),
            pl.BlockSpec((1, MP, K2_, KDIM_), lambda b: (b, 0, 0, 0)),
        ],
        out_specs=[
            pl.BlockSpec((1, K2_, MP), lambda b: (b, 0, 0)),
            pl.BlockSpec((TROWS, TW), lambda b: (b, 0)),
        ],
        out_shape=[
            jax.ShapeDtypeStruct((B, K2_, MP), jnp.int32),
            jax.ShapeDtypeStruct((B * TROWS, TW), jnp.float32),
        ],
        scratch_shapes=[pltpu.VMEM((KP_ROWS, KDIM_), jnp.float32)],
        compiler_params=pltpu.CompilerParams(
            dimension_semantics=("arbitrary",)),
    )(k, v, mk_pad)

    tot = B * MP * K2_
    ch = tot // NW
    nbr_flat = nbr.transpose(0, 2, 1).reshape(tot)

    mesh = plsc.VectorSubcoreMesh(
        core_axis_name="c", subcore_axis_name="s",
        num_cores=NC, num_subcores=NS)
    sc_gather = pl.kernel(
        functools.partial(_sc_gather_body, ch),
        out_type=jax.ShapeDtypeStruct((tot, TW), jnp.float32),
        mesh=mesh,
        scratch_types=[
            pltpu.VMEM((ch,), jnp.int32),
            pltpu.VMEM((ch, TW), jnp.float32),
            pltpu.SemaphoreType.DMA,
        ],
    )
    out_flat = sc_gather(nbr_flat, tbl)

    out4 = out_flat.reshape(B, MP, K2_, TW)
    outk = out4[:, :M_, :, :KDIM_]
    outv = out4[:, :M_, :, KDIM_:KDIM_ + VDIM_]
    return outk, outv


# speedup vs baseline: 1.1780x; 1.1780x over previous
import functools

import jax
import jax.numpy as jnp
from jax import lax
from jax.experimental import pallas as pl
from jax.experimental.pallas import tpu as pltpu
from jax.experimental.pallas import tpu_sc as plsc

M_ = 100
MP = 128
K2_ = 9
KDIM_ = 256
VDIM_ = 3
TW = 384
H_ = 64
W_ = 64
HW_ = H_ * W_
PAD = 72
KP_ROWS = HW_ + 2 * PAD
TROWS = HW_ + 8
RBLK = 512
NC = 2
NS = 16
NW = NC * NS


def _select_body(k_ref, v_ref, mk_ref, nbr_ref, tbl_ref):
    b = pl.program_id(0)
    tbl_ref[0:PAD, :] = jnp.zeros((PAD, TW), jnp.float32)
    tbl_ref[PAD + HW_:, :] = jnp.zeros((PAD, TW), jnp.float32)
    tbl_ref[PAD:PAD + HW_, 0:KDIM_] = k_ref[0]
    tbl_ref[PAD:PAD + HW_, KDIM_:TW] = jnp.zeros((HW_, TW - KDIM_),
                                                 jnp.float32)
    tbl_ref[PAD:PAD + HW_, KDIM_:KDIM_ + VDIM_] = v_ref[0]

    xcol = jax.lax.broadcasted_iota(jnp.int32, (HW_, 1), 0) % W_

    acc = jnp.zeros((HW_, MP), jnp.float32)
    for t in range(K2_):
        dy, dx = t // 3 - 1, t % 3 - 1
        start = PAD + W_ * dy + dx
        sh = tbl_ref[start:start + HW_, 0:KDIM_]
        if dx == -1:
            sh = sh * (xcol >= 1).astype(jnp.float32)
        elif dx == 1:
            sh = sh * (xcol <= W_ - 2).astype(jnp.float32)
        w = mk_ref[0, :, t, :]
        acc = acc + jax.lax.dot_general(
            sh, w, (((1,), (1,)), ((), ())),
            preferred_element_type=jnp.float32)

    mcol = jax.lax.broadcasted_iota(jnp.int32, (HW_, MP), 1)
    lm = jnp.where(mcol < M_, acc, -1e30)
    lmax = jnp.max(lm, axis=1, keepdims=True)
    denom = jnp.sum(jnp.exp(lm - lmax), axis=1, keepdims=True)
    val = 1.0 / denom
    valT = jnp.transpose(val)

    nblk = HW_ // RBLK
    cnt_blocks = []
    for i in range(nblk):
        lo, hi = i * RBLK, (i + 1) * RBLK
        vi = val[lo:hi, :]
        acc_i = jnp.zeros((RBLK, 1), jnp.float32)
        if i > 0:
            le = valT[:, :lo] <= vi
            acc_i = acc_i + jnp.sum(
                jnp.where(le, 1.0, 0.0), axis=1, keepdims=True)
        if i < nblk - 1:
            lt = valT[:, hi:] < vi
            acc_i = acc_i + jnp.sum(
                jnp.where(lt, 1.0, 0.0), axis=1, keepdims=True)
        vd = valT[:, lo:hi]
        pio = jax.lax.broadcasted_iota(jnp.int32, (RBLK, RBLK), 0)
        qio = jax.lax.broadcasted_iota(jnp.int32, (RBLK, RBLK), 1)
        le_d = jnp.where(vd <= vi, 1.0, 0.0)
        lt_d = jnp.where(vd < vi, 1.0, 0.0)
        mix = jnp.where(qio < pio, le_d, lt_d)
        acc_i = acc_i + jnp.sum(mix, axis=1, keepdims=True)
        cnt_blocks.append(acc_i)
    rank_col = jnp.concatenate(cnt_blocks, axis=0)

    r_io = jax.lax.broadcasted_iota(jnp.int32, (HW_, MP), 1).astype(
        jnp.float32)
    q_io = jax.lax.broadcasted_iota(jnp.int32, (HW_, MP), 0).astype(
        jnp.float32)
    hit = rank_col == r_io
    srcT = jnp.sum(jnp.where(hit, q_io, 0.0), axis=0, keepdims=True)
    src = srcT.astype(jnp.int32)
    xs = src % W_

    for t in range(K2_):
        dy, dx = t // 3 - 1, t % 3 - 1
        nbr = src + (PAD + W_ * dy + dx)
        if dx == -1:
            nbr = jnp.where(xs >= 1, nbr, 0)
        elif dx == 1:
            nbr = jnp.where(xs <= W_ - 2, nbr, 0)
        nbr_ref[0, pl.ds(t, 1), :] = nbr + b * KP_ROWS


def _sc_gather_body(ch, nbr_hbm, tbl_hbm, out_hbm, idx_v, rows, sem):
    wid = lax.axis_index("s") * NC + lax.axis_index("c")
    base = wid * ch
    pltpu.sync_copy(nbr_hbm.at[pl.ds(base, ch)], idx_v)
    pltpu.async_copy(tbl_hbm.at[idx_v], rows, sem).wait()
    pltpu.sync_copy(rows, out_hbm.at[pl.ds(base, ch)])


def kernel(k, v, rkn_score, m_k, m_v, m_u):
    del rkn_score, m_v, m_u
    B = k.shape[0]
    mk_pad = jnp.pad(m_k, ((0, 0), (0, MP - M_), (0, 0), (0, 0)))

    nbr, tbl = pl.pallas_call(
        _select_body,
        grid=(B,),
        in_specs=[
            pl.BlockSpec((1, HW_, KDIM_), lambda b: (b, 0, 0)),
            pl.BlockSpec((1, HW_, VDIM_), lambda b: (b, 0, 0)),
            pl.BlockSpec((1, MP, K2_, KDIM_), lambda b: (b, 0, 0, 0)),
        ],
        out_specs=[
            pl.BlockSpec((1, K2_, MP), lambda b: (b, 0, 0)),
            pl.BlockSpec((KP_ROWS, TW), lambda b: (b, 0)),
        ],
        out_shape=[
            jax.ShapeDtypeStruct((B, K2_, MP), jnp.int32),
            jax.ShapeDtypeStruct((B * KP_ROWS, TW), jnp.float32),
        ],
        compiler_params=pltpu.CompilerParams(
            dimension_semantics=("arbitrary",)),
    )(k, v, mk_pad)

    tot = B * MP * K2_
    ch = tot // NW
    nbr_flat = nbr.transpose(0, 2, 1).reshape(tot)

    mesh = plsc.VectorSubcoreMesh(
        core_axis_name="c", subcore_axis_name="s",
        num_cores=NC, num_subcores=NS)
    sc_gather = pl.kernel(
        functools.partial(_sc_gather_body, ch),
        out_type=jax.ShapeDtypeStruct((tot, TW), jnp.float32),
        mesh=mesh,
        scratch_types=[
            pltpu.VMEM((ch,), jnp.int32),
            pltpu.VMEM((ch, TW), jnp.float32),
            pltpu.SemaphoreType.DMA,
        ],
    )
    out_flat = sc_gather(nbr_flat, tbl)

    out4 = out_flat.reshape(B, MP, K2_, TW)
    outk = out4[:, :M_, :, :KDIM_]
    outv = out4[:, :M_, :, KDIM_:KDIM_ + VDIM_]
    return outk, outv
